# baseline (speedup 1.0000x reference)
import functools

import jax
import jax.numpy as jnp
import numpy as np
from jax import lax
from jax.experimental import pallas as pl
from jax.experimental.pallas import tpu as pltpu
from jax.experimental.pallas import tpu_sc as plsc

N = 10000
D = 128
C = 40
E = 320000

NC, NS = 2, 16
ROWS_PER_TILE = 632
N_PAD = NS * ROWS_PER_TILE
CHUNK = 128
STEPS_PER_TILE = 80
NB = 2
NBI = 4
E_PAD = NC * NS * STEPS_PER_TILE * CHUNK

@functools.cache
def _make_sc_aggregate():
    mesh = plsc.VectorSubcoreMesh(
        core_axis_name="c", subcore_axis_name="s", num_cores=NC, num_subcores=NS
    )

    @functools.partial(
        pl.kernel,
        out_type=jax.ShapeDtypeStruct((NC, N_PAD, D), jnp.float32),
        mesh=mesh,
        scratch_types=[
            pltpu.VMEM((NBI, CHUNK), jnp.int32),
            pltpu.VMEM((NBI, CHUNK), jnp.int32),
            pltpu.VMEM((NB, CHUNK, D), jnp.float32),
            pltpu.VMEM_SHARED((N_PAD, D), jnp.float32),
            [pltpu.SemaphoreType.DMA] * NB,
            [pltpu.SemaphoreType.DMA] * NBI,
            [pltpu.SemaphoreType.DMA] * NBI,
        ],
    )
    def sc_aggregate(h_hbm, src_hbm, dst_hbm, zeros_hbm, out_hbm,
                     src_ring, dst_ring, rows_v, agg_sh, gsems, ssems, dsems):
        c = lax.axis_index("c")
        s = lax.axis_index("s")
        row0 = s * ROWS_PER_TILE
        wid = c * NS + s
        base = wid * STEPS_PER_TILE
        pltpu.sync_copy(zeros_hbm.at[pl.ds(row0, ROWS_PER_TILE)],
                        agg_sh.at[pl.ds(row0, ROWS_PER_TILE)])
        plsc.subcore_barrier()

        for k in range(NBI):
            g = (base + k) * CHUNK
            pltpu.async_copy(src_hbm.at[pl.ds(g, CHUNK)], src_ring.at[k], ssems[k])
            pltpu.async_copy(dst_hbm.at[pl.ds(g, CHUNK)], dst_ring.at[k], dsems[k])
        for b in range(NB):
            pltpu.make_async_copy(src_hbm.at[pl.ds(0, CHUNK)], src_ring.at[b],
                                  ssems[b]).wait()
            pltpu.async_copy(h_hbm.at[src_ring.at[b]], rows_v.at[b], gsems[b])

        def outer(o, carry):
            for k in range(NBI):
                b = k % NB
                j = o * NBI + k
                pltpu.make_async_copy(h_hbm.at[src_ring.at[0]], rows_v.at[b],
                                      gsems[b]).wait()
                pltpu.make_async_copy(src_hbm.at[pl.ds(0, CHUNK)],
                                      dst_ring.at[k], dsems[k]).wait()
                pltpu.sync_copy(rows_v.at[b], agg_sh.at[dst_ring.at[k]], add=True)

                @pl.when(j + NBI < STEPS_PER_TILE)
                def _():
                    g = (base + j + NBI) * CHUNK
                    pltpu.async_copy(src_hbm.at[pl.ds(g, CHUNK)], src_ring.at[k],
                                     ssems[k])
                    pltpu.async_copy(dst_hbm.at[pl.ds(g, CHUNK)], dst_ring.at[k],
                                     dsems[k])

                kn = (k + NB) % NBI

                @pl.when(j + NB < STEPS_PER_TILE)
                def _():
                    pltpu.make_async_copy(src_hbm.at[pl.ds(0, CHUNK)],
                                          src_ring.at[kn], ssems[kn]).wait()
                    pltpu.async_copy(h_hbm.at[src_ring.at[kn]], rows_v.at[b],
                                     gsems[b])
            return carry

        lax.fori_loop(0, STEPS_PER_TILE // NBI, outer, 0)
        plsc.subcore_barrier()
        pltpu.sync_copy(agg_sh.at[pl.ds(row0, ROWS_PER_TILE)],
                        out_hbm.at[c, pl.ds(row0, ROWS_PER_TILE)])

    return sc_aggregate


def _sc_aggregate(h, src_p, dst_p, zeros):
    return _make_sc_aggregate()(h, src_p, dst_p, zeros)


BM = 2000


def _elu(z):
    return jnp.where(z > 0.0, z, jnp.exp(z) - 1.0)


def _mlp_block(h_ref, agg_ref, w1_ref, b1_ref, w2_ref, b2_ref):
    z = h_ref[...] + agg_ref[0] + agg_ref[1]
    z = _elu(jnp.dot(z, w1_ref[...], preferred_element_type=jnp.float32,
                     precision=lax.Precision.HIGHEST) + b1_ref[...])
    z = _elu(jnp.dot(z, w2_ref[...], preferred_element_type=jnp.float32,
                     precision=lax.Precision.HIGHEST) + b2_ref[...])
    return z


def _tc_mlp_body(h_ref, agg_ref, w1_ref, b1_ref, w2_ref, b2_ref, out_ref):
    out_ref[...] = _mlp_block(h_ref, agg_ref, w1_ref, b1_ref, w2_ref, b2_ref)


def _tc_mlp_fc_body(h_ref, agg_ref, w1_ref, b1_ref, w2_ref, b2_ref,
                    wfc_ref, bfc_ref, out_ref):
    z = _mlp_block(h_ref, agg_ref, w1_ref, b1_ref, w2_ref, b2_ref)
    out_ref[...] = jnp.dot(z, wfc_ref[...], preferred_element_type=jnp.float32,
                           precision=lax.Precision.HIGHEST) + bfc_ref[...]


def _full(shape):
    return pl.BlockSpec(shape, lambda i: tuple(0 for _ in shape))


_H_SPEC = pl.BlockSpec((BM, D), lambda i: (i, 0))
_AGG_SPEC = pl.BlockSpec((NC, BM, D), lambda i: (0, i, 0))


def _tc_mlp(h, agg, w1, b1, w2, b2):
    return pl.pallas_call(
        _tc_mlp_body,
        grid=(N // BM,),
        in_specs=[_H_SPEC, _AGG_SPEC, _full((D, D)), _full((1, D)),
                  _full((D, D)), _full((1, D))],
        out_specs=_H_SPEC,
        out_shape=jax.ShapeDtypeStruct((N, D), jnp.float32),
    )(h, agg, w1, b1, w2, b2)


def _tc_mlp_fc(h, agg, w1, b1, w2, b2, wfc, bfc):
    return pl.pallas_call(
        _tc_mlp_fc_body,
        grid=(N // BM,),
        in_specs=[_H_SPEC, _AGG_SPEC, _full((D, D)), _full((1, D)),
                  _full((D, D)), _full((1, D)), _full((D, C)), _full((1, C))],
        out_specs=pl.BlockSpec((BM, C), lambda i: (i, 0)),
        out_shape=jax.ShapeDtypeStruct((N, C), jnp.float32),
    )(h, agg, w1, b1, w2, b2, wfc, bfc)


def kernel(x, edge_index, weight, W1_0, b1_0, W2_0, b2_0, W1_1, b1_1, W2_1,
           b2_1, W1_2, b1_2, W2_2, b2_2, Wfc, bfc):
    del weight
    ar = np.arange(E_PAD - E, dtype=np.int32)
    src_p = jnp.concatenate([edge_index[0], jnp.asarray(ar % N)])
    dst_p = jnp.concatenate([edge_index[1], jnp.asarray(N + ar % (N_PAD - N))])
    zeros = jnp.zeros((N_PAD, D), jnp.float32)

    h = x
    for (w1, b1, w2, b2) in ((W1_0, b1_0, W2_0, b2_0), (W1_1, b1_1, W2_1, b2_1)):
        agg = _sc_aggregate(h, src_p, dst_p, zeros)
        h = _tc_mlp(h, agg, w1.reshape(D, D), b1.reshape(1, D),
                    w2.reshape(D, D), b2.reshape(1, D))
    agg = _sc_aggregate(h, src_p, dst_p, zeros)
    out = _tc_mlp_fc(h, agg, W1_2, b1_2.reshape(1, D),
                     W2_2, b2_2.reshape(1, D), Wfc, bfc.reshape(1, C))
    return out


# speedup vs baseline: 1.0867x; 1.0867x over previous
import functools

import jax
import jax.numpy as jnp
import numpy as np
from jax import lax
from jax.experimental import pallas as pl
from jax.experimental.pallas import tpu as pltpu
from jax.experimental.pallas import tpu_sc as plsc

N = 10000
D = 128
C = 40
E = 320000

NC, NS = 2, 16
ROWS_PER_TILE = 632
N_PAD = NS * ROWS_PER_TILE
CHUNK = 128
STEPS_PER_TILE = 80
NB = 2
NBI = 4
E_PAD = NC * NS * STEPS_PER_TILE * CHUNK

@functools.cache
def _make_sc_aggregate():
    mesh = plsc.VectorSubcoreMesh(
        core_axis_name="c", subcore_axis_name="s", num_cores=NC, num_subcores=NS
    )

    @functools.partial(
        pl.kernel,
        out_type=jax.ShapeDtypeStruct((NC, N_PAD, D), jnp.float32),
        mesh=mesh,
        scratch_types=[
            pltpu.VMEM((NBI, CHUNK), jnp.int32),
            pltpu.VMEM((NBI, CHUNK), jnp.int32),
            pltpu.VMEM((NB, CHUNK, D), jnp.float32),
            pltpu.VMEM_SHARED((N_PAD, D), jnp.float32),
            [pltpu.SemaphoreType.DMA] * NB,
            [pltpu.SemaphoreType.DMA] * NBI,
            [pltpu.SemaphoreType.DMA] * NBI,
        ],
    )
    def sc_aggregate(h_hbm, src_hbm, dst_hbm, zeros_hbm, out_hbm,
                     src_ring, dst_ring, rows_v, agg_sh, gsems, ssems, dsems):
        c = lax.axis_index("c")
        s = lax.axis_index("s")
        row0 = s * ROWS_PER_TILE
        wid = c * NS + s
        base = wid * STEPS_PER_TILE
        pltpu.sync_copy(zeros_hbm.at[pl.ds(row0, ROWS_PER_TILE)],
                        agg_sh.at[pl.ds(row0, ROWS_PER_TILE)])
        plsc.subcore_barrier()

        for k in range(NBI):
            g = (base + k) * CHUNK
            pltpu.async_copy(src_hbm.at[pl.ds(g, CHUNK)], src_ring.at[k], ssems[k])
            pltpu.async_copy(dst_hbm.at[pl.ds(g, CHUNK)], dst_ring.at[k], dsems[k])
        for b in range(NB):
            pltpu.make_async_copy(src_hbm.at[pl.ds(0, CHUNK)], src_ring.at[b],
                                  ssems[b]).wait()
            pltpu.async_copy(h_hbm.at[src_ring.at[b]], rows_v.at[b], gsems[b])

        def outer(o, carry):
            for k in range(NBI):
                b = k % NB
                j = o * NBI + k
                pltpu.make_async_copy(h_hbm.at[src_ring.at[0]], rows_v.at[b],
                                      gsems[b]).wait()
                pltpu.make_async_copy(src_hbm.at[pl.ds(0, CHUNK)],
                                      dst_ring.at[k], dsems[k]).wait()
                pltpu.sync_copy(rows_v.at[b], agg_sh.at[dst_ring.at[k]], add=True)

                @pl.when(j + NBI < STEPS_PER_TILE)
                def _():
                    g = (base + j + NBI) * CHUNK
                    pltpu.async_copy(src_hbm.at[pl.ds(g, CHUNK)], src_ring.at[k],
                                     ssems[k])
                    pltpu.async_copy(dst_hbm.at[pl.ds(g, CHUNK)], dst_ring.at[k],
                                     dsems[k])

                kn = (k + NB) % NBI

                @pl.when(j + NB < STEPS_PER_TILE)
                def _():
                    pltpu.make_async_copy(src_hbm.at[pl.ds(0, CHUNK)],
                                          src_ring.at[kn], ssems[kn]).wait()
                    pltpu.async_copy(h_hbm.at[src_ring.at[kn]], rows_v.at[b],
                                     gsems[b])
            return carry

        lax.fori_loop(0, STEPS_PER_TILE // NBI, outer, 0)
        plsc.subcore_barrier()
        pltpu.sync_copy(agg_sh.at[pl.ds(row0, ROWS_PER_TILE)],
                        out_hbm.at[c, pl.ds(row0, ROWS_PER_TILE)])

    return sc_aggregate


def _sc_aggregate(h, src_p, dst_p, zeros):
    return _make_sc_aggregate()(h, src_p, dst_p, zeros)


BM = 2000


def _elu(z):
    return jnp.where(z > 0.0, z, jnp.exp(z) - 1.0)


def _mlp_block(h_ref, agg_ref, w1_ref, b1_ref, w2_ref, b2_ref):
    z = h_ref[...] + agg_ref[0] + agg_ref[1]
    z = _elu(jnp.dot(z, w1_ref[...], preferred_element_type=jnp.float32)
             + b1_ref[...])
    z = _elu(jnp.dot(z, w2_ref[...], preferred_element_type=jnp.float32)
             + b2_ref[...])
    return z


def _tc_mlp_body(h_ref, agg_ref, w1_ref, b1_ref, w2_ref, b2_ref, out_ref):
    out_ref[...] = _mlp_block(h_ref, agg_ref, w1_ref, b1_ref, w2_ref, b2_ref)


def _tc_mlp_fc_body(h_ref, agg_ref, w1_ref, b1_ref, w2_ref, b2_ref,
                    wfc_ref, bfc_ref, out_ref):
    z = _mlp_block(h_ref, agg_ref, w1_ref, b1_ref, w2_ref, b2_ref)
    out_ref[...] = jnp.dot(z, wfc_ref[...], preferred_element_type=jnp.float32
                           ) + bfc_ref[...]


def _full(shape):
    return pl.BlockSpec(shape, lambda i: tuple(0 for _ in shape))


_H_SPEC = pl.BlockSpec((BM, D), lambda i: (i, 0))
_AGG_SPEC = pl.BlockSpec((NC, BM, D), lambda i: (0, i, 0))


def _tc_mlp(h, agg, w1, b1, w2, b2):
    return pl.pallas_call(
        _tc_mlp_body,
        grid=(N // BM,),
        in_specs=[_H_SPEC, _AGG_SPEC, _full((D, D)), _full((1, D)),
                  _full((D, D)), _full((1, D))],
        out_specs=_H_SPEC,
        out_shape=jax.ShapeDtypeStruct((N, D), jnp.float32),
    )(h, agg, w1, b1, w2, b2)


def _tc_mlp_fc(h, agg, w1, b1, w2, b2, wfc, bfc):
    return pl.pallas_call(
        _tc_mlp_fc_body,
        grid=(N // BM,),
        in_specs=[_H_SPEC, _AGG_SPEC, _full((D, D)), _full((1, D)),
                  _full((D, D)), _full((1, D)), _full((D, C)), _full((1, C))],
        out_specs=pl.BlockSpec((BM, C), lambda i: (i, 0)),
        out_shape=jax.ShapeDtypeStruct((N, C), jnp.float32),
    )(h, agg, w1, b1, w2, b2, wfc, bfc)


def kernel(x, edge_index, weight, W1_0, b1_0, W2_0, b2_0, W1_1, b1_1, W2_1,
           b2_1, W1_2, b1_2, W2_2, b2_2, Wfc, bfc):
    del weight
    ar = np.arange(E_PAD - E, dtype=np.int32)
    src_p = jnp.concatenate([edge_index[0], jnp.asarray(ar % N)])
    dst_p = jnp.concatenate([edge_index[1], jnp.asarray(N + ar % (N_PAD - N))])
    zeros = jnp.zeros((N_PAD, D), jnp.float32)

    h = x
    for (w1, b1, w2, b2) in ((W1_0, b1_0, W2_0, b2_0), (W1_1, b1_1, W2_1, b2_1)):
        agg = _sc_aggregate(h, src_p, dst_p, zeros)
        h = _tc_mlp(h, agg, w1.reshape(D, D), b1.reshape(1, D),
                    w2.reshape(D, D), b2.reshape(1, D))
    agg = _sc_aggregate(h, src_p, dst_p, zeros)
    out = _tc_mlp_fc(h, agg, W1_2, b1_2.reshape(1, D),
                     W2_2, b2_2.reshape(1, D), Wfc, bfc.reshape(1, C))
    return out


# speedup vs baseline: 1.0984x; 1.0108x over previous
import functools

import jax
import jax.numpy as jnp
import numpy as np
from jax import lax
from jax.experimental import pallas as pl
from jax.experimental.pallas import tpu as pltpu
from jax.experimental.pallas import tpu_sc as plsc

N = 10000
D = 128
C = 40
E = 320000

NC, NS = 2, 16
ROWS_PER_TILE = 632
N_PAD = NS * ROWS_PER_TILE
CHUNK = 128
STEPS_PER_TILE = 80
NB = 2
NBI = 4
E_PAD = NC * NS * STEPS_PER_TILE * CHUNK

@functools.cache
def _make_sc_aggregate():
    mesh = plsc.VectorSubcoreMesh(
        core_axis_name="c", subcore_axis_name="s", num_cores=NC, num_subcores=NS
    )

    @functools.partial(
        pl.kernel,
        out_type=jax.ShapeDtypeStruct((NC, N_PAD, D), jnp.float32),
        mesh=mesh,
        scratch_types=[
            pltpu.VMEM((NBI, CHUNK), jnp.int32),
            pltpu.VMEM((NBI, CHUNK), jnp.int32),
            pltpu.VMEM((NB, CHUNK, D), jnp.float32),
            pltpu.VMEM_SHARED((N_PAD, D), jnp.float32),
            [pltpu.SemaphoreType.DMA] * NB,
            [pltpu.SemaphoreType.DMA] * NBI,
            [pltpu.SemaphoreType.DMA] * NBI,
        ],
    )
    def sc_aggregate(h_hbm, src_hbm, dst_hbm, zeros_hbm, out_hbm,
                     src_ring, dst_ring, rows_v, agg_sh, gsems, ssems, dsems):
        c = lax.axis_index("c")
        s = lax.axis_index("s")
        row0 = s * ROWS_PER_TILE
        wid = c * NS + s
        base = wid * STEPS_PER_TILE
        for k in range(NBI):
            g = (base + k) * CHUNK
            pltpu.async_copy(src_hbm.at[pl.ds(g, CHUNK)], src_ring.at[k], ssems[k])
            pltpu.async_copy(dst_hbm.at[pl.ds(g, CHUNK)], dst_ring.at[k], dsems[k])
        pltpu.sync_copy(zeros_hbm.at[pl.ds(row0, ROWS_PER_TILE)],
                        agg_sh.at[pl.ds(row0, ROWS_PER_TILE)])
        for b in range(NB):
            pltpu.make_async_copy(src_hbm.at[pl.ds(0, CHUNK)], src_ring.at[b],
                                  ssems[b]).wait()
            pltpu.async_copy(h_hbm.at[src_ring.at[b]], rows_v.at[b], gsems[b])
        plsc.subcore_barrier()

        def outer(o, carry):
            for k in range(NBI):
                b = k % NB
                j = o * NBI + k
                pltpu.make_async_copy(h_hbm.at[src_ring.at[0]], rows_v.at[b],
                                      gsems[b]).wait()
                pltpu.make_async_copy(src_hbm.at[pl.ds(0, CHUNK)],
                                      dst_ring.at[k], dsems[k]).wait()
                pltpu.sync_copy(rows_v.at[b], agg_sh.at[dst_ring.at[k]], add=True)

                @pl.when(j + NBI < STEPS_PER_TILE)
                def _():
                    g = (base + j + NBI) * CHUNK
                    pltpu.async_copy(src_hbm.at[pl.ds(g, CHUNK)], src_ring.at[k],
                                     ssems[k])
                    pltpu.async_copy(dst_hbm.at[pl.ds(g, CHUNK)], dst_ring.at[k],
                                     dsems[k])

                kn = (k + NB) % NBI

                @pl.when(j + NB < STEPS_PER_TILE)
                def _():
                    pltpu.make_async_copy(src_hbm.at[pl.ds(0, CHUNK)],
                                          src_ring.at[kn], ssems[kn]).wait()
                    pltpu.async_copy(h_hbm.at[src_ring.at[kn]], rows_v.at[b],
                                     gsems[b])
            return carry

        lax.fori_loop(0, STEPS_PER_TILE // NBI, outer, 0)
        plsc.subcore_barrier()
        pltpu.sync_copy(agg_sh.at[pl.ds(row0, ROWS_PER_TILE)],
                        out_hbm.at[c, pl.ds(row0, ROWS_PER_TILE)])

    return sc_aggregate


def _sc_aggregate(h, src_p, dst_p, zeros):
    return _make_sc_aggregate()(h, src_p, dst_p, zeros)


BM = 2000


def _elu(z):
    return jnp.where(z > 0.0, z, jnp.exp(z) - 1.0)


def _mlp_block(h_ref, agg_ref, w1_ref, b1_ref, w2_ref, b2_ref):
    z = h_ref[...] + agg_ref[0] + agg_ref[1]
    z = _elu(jnp.dot(z, w1_ref[...], preferred_element_type=jnp.float32)
             + b1_ref[...])
    z = _elu(jnp.dot(z, w2_ref[...], preferred_element_type=jnp.float32)
             + b2_ref[...])
    return z


def _tc_mlp_body(h_ref, agg_ref, w1_ref, b1_ref, w2_ref, b2_ref, out_ref):
    out_ref[...] = _mlp_block(h_ref, agg_ref, w1_ref, b1_ref, w2_ref, b2_ref)


def _tc_mlp_fc_body(h_ref, agg_ref, w1_ref, b1_ref, w2_ref, b2_ref,
                    wfc_ref, bfc_ref, out_ref):
    z = _mlp_block(h_ref, agg_ref, w1_ref, b1_ref, w2_ref, b2_ref)
    out_ref[...] = jnp.dot(z, wfc_ref[...], preferred_element_type=jnp.float32
                           ) + bfc_ref[...]


def _full(shape):
    return pl.BlockSpec(shape, lambda i: tuple(0 for _ in shape))


_H_SPEC = pl.BlockSpec((BM, D), lambda i: (i, 0))
_AGG_SPEC = pl.BlockSpec((NC, BM, D), lambda i: (0, i, 0))


def _tc_mlp(h, agg, w1, b1, w2, b2):
    return pl.pallas_call(
        _tc_mlp_body,
        grid=(N // BM,),
        in_specs=[_H_SPEC, _AGG_SPEC, _full((D, D)), _full((1, D)),
                  _full((D, D)), _full((1, D))],
        out_specs=_H_SPEC,
        out_shape=jax.ShapeDtypeStruct((N, D), jnp.float32),
    )(h, agg, w1, b1, w2, b2)


def _tc_mlp_fc(h, agg, w1, b1, w2, b2, wfc, bfc):
    return pl.pallas_call(
        _tc_mlp_fc_body,
        grid=(N // BM,),
        in_specs=[_H_SPEC, _AGG_SPEC, _full((D, D)), _full((1, D)),
                  _full((D, D)), _full((1, D)), _full((D, C)), _full((1, C))],
        out_specs=pl.BlockSpec((BM, C), lambda i: (i, 0)),
        out_shape=jax.ShapeDtypeStruct((N, C), jnp.float32),
    )(h, agg, w1, b1, w2, b2, wfc, bfc)


def kernel(x, edge_index, weight, W1_0, b1_0, W2_0, b2_0, W1_1, b1_1, W2_1,
           b2_1, W1_2, b1_2, W2_2, b2_2, Wfc, bfc):
    del weight
    ar = np.arange(E_PAD - E, dtype=np.int32)
    src_p = jnp.concatenate([edge_index[0], jnp.asarray(ar % N)])
    dst_p = jnp.concatenate([edge_index[1], jnp.asarray(N + ar % (N_PAD - N))])
    zeros = jnp.asarray(np.zeros((N_PAD, D), np.float32))

    h = x
    for (w1, b1, w2, b2) in ((W1_0, b1_0, W2_0, b2_0), (W1_1, b1_1, W2_1, b2_1)):
        agg = _sc_aggregate(h, src_p, dst_p, zeros)
        h = _tc_mlp(h, agg, w1.reshape(D, D), b1.reshape(1, D),
                    w2.reshape(D, D), b2.reshape(1, D))
    agg = _sc_aggregate(h, src_p, dst_p, zeros)
    out = _tc_mlp_fc(h, agg, W1_2, b1_2.reshape(1, D),
                     W2_2, b2_2.reshape(1, D), Wfc, bfc.reshape(1, C))
    return out
